# Initial kernel scaffold; baseline (speedup 1.0000x reference)
import jax
import jax.numpy as jnp
from jax.experimental import pallas as pl


def kernel(node_features, edge_index, edge_weight, W1, b1, W2, b2):
    raise NotImplementedError("write your pallas kernel here")


# speedup vs baseline: 6.7475x; 6.7475x over previous
import functools

import jax
import jax.numpy as jnp
from jax import lax
from jax.experimental import pallas as pl
from jax.experimental.pallas import tpu as pltpu
from jax.experimental.pallas import tpu_sc as plsc

N = 10000
NPAD = 10240
E = 320000
IN_DIM = 128
HID = 256
OUT_DIM = 128

NC = 2
NS = 16
EB = 128
EP = ((E + NC * NS * EB - 1) // (NC * NS * EB)) * (NC * NS * EB)
RB = 512
NRB = NPAD // RB
NPT = NPAD // NS
NPC = NPT // 5

_mesh = plsc.VectorSubcoreMesh(core_axis_name="c", subcore_axis_name="s")


@functools.partial(
    pl.kernel,
    out_type=jax.ShapeDtypeStruct((NC, NPAD), jnp.float32),
    mesh=_mesh,
    scratch_types=[
        pltpu.VMEM((1, EB), jnp.int32),
        pltpu.VMEM((EB,), jnp.float32),
        pltpu.VMEM((NPC,), jnp.float32),
        pltpu.VMEM_SHARED((NPAD,), jnp.float32),
    ],
    compiler_params=pltpu.CompilerParams(needs_layout_passes=False),
)
def _deg_kernel(dst_hbm, ew_hbm, out_hbm, dsti_v, ew_v, zb_v, acc_sh):
    c = lax.axis_index("c")
    s = lax.axis_index("s")
    zeros16 = jnp.zeros((16,), jnp.float32)

    @pl.loop(0, NPC // 16)
    def _(r):
        zb_v[pl.ds(r * 16, 16)] = zeros16

    for k in range(NPT // NPC):
        pltpu.sync_copy(zb_v, acc_sh.at[pl.ds(s * NPT + k * NPC, NPC)])
    plsc.subcore_barrier()

    epw = EP // (NC * NS)
    nb = epw // EB
    wid = c * NS + s

    @pl.loop(0, nb)
    def _(j):
        base = wid * epw + j * EB
        pltpu.sync_copy(dst_hbm.at[pl.ds(base, EB)], dsti_v.at[0])
        pltpu.sync_copy(ew_hbm.at[pl.ds(base, EB)], ew_v)
        pltpu.sync_copy(ew_v, acc_sh.at[dsti_v.at[0]], add=True)

    plsc.subcore_barrier()
    for k in range(NPT // NPC):
        sl = pl.ds(s * NPT + k * NPC, NPC)
        pltpu.sync_copy(acc_sh.at[sl], out_hbm.at[c, sl])


def _make_agg_kernel(fh):
    @functools.partial(
        pl.kernel,
        out_type=(
            jax.ShapeDtypeStruct((NPAD, fh), jnp.float32),
            jax.ShapeDtypeStruct((NPAD, fh), jnp.float32),
        ),
        mesh=_mesh,
        scratch_types=[
            pltpu.VMEM((1, EB), jnp.int32),
            pltpu.VMEM((1, EB), jnp.int32),
            pltpu.VMEM((EB,), jnp.float32),
            pltpu.VMEM((EB, fh), jnp.float32),
            pltpu.VMEM((NPC, fh), jnp.float32),
            pltpu.VMEM_SHARED((NPAD, fh), jnp.float32),
        ],
        compiler_params=pltpu.CompilerParams(needs_layout_passes=False),
    )
    def agg(ha_hbm, hb_hbm, src_hbm, dst_hbm, ew_hbm, oa_hbm, ob_hbm,
            srci_v, dsti_v, ew_v, rows_v, zb_v, acc_sh):
        c = lax.axis_index("c")
        s = lax.axis_index("s")
        zeros16 = jnp.zeros((16,), jnp.float32)

        @pl.loop(0, NPC)
        def _(r):
            for g in range(fh // 16):
                zb_v[r, pl.ds(g * 16, 16)] = zeros16

        for k in range(NPT // NPC):
            pltpu.sync_copy(zb_v, acc_sh.at[pl.ds(s * NPT + k * NPC, NPC)])
        plsc.subcore_barrier()

        ept = EP // NS
        nb = ept // EB

        @pl.loop(0, nb)
        def _(j):
            base = s * ept + j * EB
            pltpu.sync_copy(src_hbm.at[pl.ds(base, EB)], srci_v.at[0])
            pltpu.sync_copy(dst_hbm.at[pl.ds(base, EB)], dsti_v.at[0])
            pltpu.sync_copy(ew_hbm.at[pl.ds(base, EB)], ew_v)

            @pl.when(c == 0)
            def _():
                pltpu.sync_copy(ha_hbm.at[srci_v.at[0]], rows_v)

            @pl.when(c == 1)
            def _():
                pltpu.sync_copy(hb_hbm.at[srci_v.at[0]], rows_v)

            @pl.loop(0, EB)
            def _(r):
                w = plsc.load_gather(ew_v, [jnp.full((16,), r, jnp.int32)])
                for g in range(fh // 16):
                    sl = pl.ds(g * 16, 16)
                    rows_v[r, sl] = rows_v[r, sl] * w

            pltpu.sync_copy(rows_v, acc_sh.at[dsti_v.at[0]], add=True)

        plsc.subcore_barrier()
        for k in range(NPT // NPC):
            sl = pl.ds(s * NPT + k * NPC, NPC)

            @pl.when(c == 0)
            def _():
                pltpu.sync_copy(acc_sh.at[sl], oa_hbm.at[sl])

            @pl.when(c == 1)
            def _():
                pltpu.sync_copy(acc_sh.at[sl], ob_hbm.at[sl])

    return agg


_agg1 = _make_agg_kernel(HID // 2)


@functools.partial(
    pl.kernel,
    out_type=(
        jax.ShapeDtypeStruct((NPAD, OUT_DIM), jnp.float32),
        jax.ShapeDtypeStruct((NPAD, OUT_DIM), jnp.float32),
    ),
    mesh=_mesh,
    scratch_types=[
        pltpu.VMEM((1, EB), jnp.int32),
        pltpu.VMEM((1, EB), jnp.int32),
        pltpu.VMEM((EB,), jnp.float32),
        pltpu.VMEM((EB, OUT_DIM), jnp.float32),
        pltpu.VMEM((NPC, OUT_DIM), jnp.float32),
        pltpu.VMEM_SHARED((NPAD, OUT_DIM), jnp.float32),
    ],
    compiler_params=pltpu.CompilerParams(needs_layout_passes=False),
)
def _agg2(h_hbm, src_hbm, dst_hbm, ew_hbm, o0_hbm, o1_hbm,
          srci_v, dsti_v, ew_v, rows_v, zb_v, acc_sh):
    c = lax.axis_index("c")
    s = lax.axis_index("s")
    zeros16 = jnp.zeros((16,), jnp.float32)

    @pl.loop(0, NPC)
    def _(r):
        for g in range(OUT_DIM // 16):
            zb_v[r, pl.ds(g * 16, 16)] = zeros16

    for k in range(NPT // NPC):
        pltpu.sync_copy(zb_v, acc_sh.at[pl.ds(s * NPT + k * NPC, NPC)])
    plsc.subcore_barrier()

    epw = EP // (NC * NS)
    nb = epw // EB
    wid = c * NS + s

    @pl.loop(0, nb)
    def _(j):
        base = wid * epw + j * EB
        pltpu.sync_copy(src_hbm.at[pl.ds(base, EB)], srci_v.at[0])
        pltpu.sync_copy(dst_hbm.at[pl.ds(base, EB)], dsti_v.at[0])
        pltpu.sync_copy(ew_hbm.at[pl.ds(base, EB)], ew_v)
        pltpu.sync_copy(h_hbm.at[srci_v.at[0]], rows_v)

        @pl.loop(0, EB)
        def _(r):
            w = plsc.load_gather(ew_v, [jnp.full((16,), r, jnp.int32)])
            for g in range(OUT_DIM // 16):
                sl = pl.ds(g * 16, 16)
                rows_v[r, sl] = rows_v[r, sl] * w

        pltpu.sync_copy(rows_v, acc_sh.at[dsti_v.at[0]], add=True)

    plsc.subcore_barrier()
    for k in range(NPT // NPC):
        sl = pl.ds(s * NPT + k * NPC, NPC)

        @pl.when(c == 0)
        def _():
            pltpu.sync_copy(acc_sh.at[sl], o0_hbm.at[sl])

        @pl.when(c == 1)
        def _():
            pltpu.sync_copy(acc_sh.at[sl], o1_hbm.at[sl])


def _mm1_body(degp_ref, x_ref, w1_ref, h1a_ref, h1b_ref, dinv_ref):
    i = pl.program_id(0)
    deg = degp_ref[0, i, :] + degp_ref[1, i, :] + 1.0
    dinv = lax.rsqrt(deg)
    dinv_ref[...] = dinv[None, None, :]
    h = jnp.dot(x_ref[...], w1_ref[...], preferred_element_type=jnp.float32)
    hs = h * dinv[:, None]
    h1a_ref[...] = hs[:, : HID // 2]
    h1b_ref[...] = hs[:, HID // 2:]


def _mm1(degp, x, w1):
    return pl.pallas_call(
        _mm1_body,
        grid=(NRB,),
        in_specs=[
            pl.BlockSpec((NC, NRB, RB), lambda i: (0, 0, 0)),
            pl.BlockSpec((RB, IN_DIM), lambda i: (i, 0)),
            pl.BlockSpec((IN_DIM, HID), lambda i: (0, 0)),
        ],
        out_specs=[
            pl.BlockSpec((RB, HID // 2), lambda i: (i, 0)),
            pl.BlockSpec((RB, HID // 2), lambda i: (i, 0)),
            pl.BlockSpec((1, 1, RB), lambda i: (i, 0, 0)),
        ],
        out_shape=[
            jax.ShapeDtypeStruct((NPAD, HID // 2), jnp.float32),
            jax.ShapeDtypeStruct((NPAD, HID // 2), jnp.float32),
            jax.ShapeDtypeStruct((NRB, 1, RB), jnp.float32),
        ],
    )(degp, x, w1)


def _mid_body(a1a_ref, a1b_ref, h1a_ref, h1b_ref, dinv_ref, b1_ref, w2_ref,
              h2_ref):
    dinv = dinv_ref[0, 0, :]
    t = jnp.concatenate(
        [a1a_ref[...] + h1a_ref[...], a1b_ref[...] + h1b_ref[...]], axis=1)
    t = jnp.maximum(t * dinv[:, None] + b1_ref[...][None, :], 0.0)
    h2 = jnp.dot(t, w2_ref[...], preferred_element_type=jnp.float32)
    h2_ref[...] = h2 * dinv[:, None]


def _mid(a1a, a1b, h1a, h1b, dinv, b1, w2):
    half = pl.BlockSpec((RB, HID // 2), lambda i: (i, 0))
    return pl.pallas_call(
        _mid_body,
        grid=(NRB,),
        in_specs=[
            half, half, half, half,
            pl.BlockSpec((1, 1, RB), lambda i: (i, 0, 0)),
            pl.BlockSpec((HID,), lambda i: (0,)),
            pl.BlockSpec((HID, OUT_DIM), lambda i: (0, 0)),
        ],
        out_specs=pl.BlockSpec((RB, OUT_DIM), lambda i: (i, 0)),
        out_shape=jax.ShapeDtypeStruct((NPAD, OUT_DIM), jnp.float32),
    )(a1a, a1b, h1a, h1b, dinv, b1, w2)


def _fin_body(p0_ref, p1_ref, h2_ref, dinv_ref, b2_ref, out_ref):
    dinv = dinv_ref[0, 0, :]
    o = p0_ref[...] + p1_ref[...] + h2_ref[...]
    out_ref[...] = o * dinv[:, None] + b2_ref[...][None, :]


def _fin(p0, p1, h2, dinv, b2):
    full = pl.BlockSpec((RB, OUT_DIM), lambda i: (i, 0))
    return pl.pallas_call(
        _fin_body,
        grid=(NRB,),
        in_specs=[
            full, full, full,
            pl.BlockSpec((1, 1, RB), lambda i: (i, 0, 0)),
            pl.BlockSpec((OUT_DIM,), lambda i: (0,)),
        ],
        out_specs=full,
        out_shape=jax.ShapeDtypeStruct((NPAD, OUT_DIM), jnp.float32),
    )(p0, p1, h2, dinv, b2)


def kernel(node_features, edge_index, edge_weight, W1, b1, W2, b2):
    pad = EP - E
    src = jnp.pad(edge_index[0], (0, pad))
    dst = jnp.pad(edge_index[1], (0, pad))
    ew = jnp.pad(edge_weight, (0, pad))

    xpad = jnp.pad(node_features, ((0, NPAD - N), (0, 0)))

    degp = _deg_kernel(dst, ew)
    degp = degp.reshape(NC, NRB, RB)
    h1a, h1b, dinv = _mm1(degp, xpad, W1)
    a1a, a1b = _agg1(h1a, h1b, src, dst, ew)
    h2 = _mid(a1a, a1b, h1a, h1b, dinv, b1, W2)
    p0, p1 = _agg2(h2, src, dst, ew)
    return _fin(p0, p1, h2, dinv, b2)[:N]
